# baseline (speedup 1.0000x reference)
import functools

import jax
import jax.numpy as jnp
from jax import lax
from jax.experimental import pallas as pl
from jax.experimental.pallas import tpu as pltpu
from jax.experimental.pallas import tpu_sc as plsc

VOCAB = 1000000
DIM = 64
WDIM = 128
BATCH = 4096
HIST = 200

B = BATCH * HIST
CHUNK = 128
SUB = 1
GROUP = CHUNK * SUB
L = 16


def _gather_kernel(num_workers):
    b_per_w = B // num_workers
    groups = b_per_w // GROUP
    pairs = groups // 2

    mesh = plsc.VectorSubcoreMesh(core_axis_name="c", subcore_axis_name="s")

    @functools.partial(
        pl.kernel,
        mesh=mesh,
        out_type=jax.ShapeDtypeStruct((B, DIM), jnp.float32),
        scratch_types=[
            pltpu.VMEM((b_per_w,), jnp.int32),
            pltpu.VMEM((GROUP, WDIM), jnp.float32),
            pltpu.VMEM((GROUP, WDIM), jnp.float32),
            pltpu.VMEM((GROUP, DIM), jnp.float32),
            pltpu.VMEM((GROUP, DIM), jnp.float32),
            pltpu.SemaphoreType.DMA,
            pltpu.SemaphoreType.DMA,
            pltpu.SemaphoreType.DMA,
            pltpu.SemaphoreType.DMA,
        ],
    )
    def gather_kernel(idx_hbm, wide_hbm, out_hbm, idx_v, rows0, rows1,
                      nar0, nar1, sem_g0, sem_g1, sem_o0, sem_o1):
        num_cores = lax.axis_size("c")
        wid = lax.axis_index("s") * num_cores + lax.axis_index("c")
        row_base = wid * b_per_w

        pltpu.sync_copy(idx_hbm.at[pl.ds(row_base, b_per_w)], idx_v)

        def fire_gather(g, rows_v, sem):
            for j in range(SUB):
                pltpu.async_copy(
                    wide_hbm.at[idx_v.at[pl.ds(g * GROUP + j * CHUNK, CHUNK)]],
                    rows_v.at[pl.ds(j * CHUNK, CHUNK)],
                    sem,
                )

        def wait_gather(rows_v, sem):
            pltpu.make_async_copy(wide_hbm.at[pl.ds(0, GROUP)], rows_v, sem).wait()

        def wait_out(nar_v, sem):
            pltpu.make_async_copy(
                out_hbm.at[pl.ds(0, GROUP)], nar_v, sem).wait()

        def compact(rows_v, nar_v):
            def body(i, carry):
                for u in range(16):
                    r = 16 * i + u
                    for q in range(DIM // L):
                        nar_v[r, pl.ds(q * L, L)] = rows_v[r, pl.ds(q * L, L)]
                return carry
            lax.fori_loop(0, GROUP // 16, body, 0)

        def fire_out(g, nar_v, sem):
            pltpu.async_copy(
                nar_v, out_hbm.at[pl.ds(row_base + g * GROUP, GROUP)], sem
            )

        fire_gather(0, rows0, sem_g0)

        def body(k, carry):
            g0 = 2 * k

            fire_gather(g0 + 1, rows1, sem_g1)

            wait_gather(rows0, sem_g0)

            @pl.when(k > 0)
            def _():
                wait_out(nar0, sem_o0)

            compact(rows0, nar0)
            fire_out(g0, nar0, sem_o0)

            g_next = lax.min(g0 + 2, groups - 1)
            fire_gather(g_next, rows0, sem_g0)

            wait_gather(rows1, sem_g1)

            @pl.when(k > 0)
            def _():
                wait_out(nar1, sem_o1)

            compact(rows1, nar1)
            fire_out(g0 + 1, nar1, sem_o1)
            return carry

        lax.fori_loop(0, pairs, body, 0)

        wait_gather(rows0, sem_g0)
        wait_out(nar0, sem_o0)
        wait_out(nar1, sem_o1)

    return gather_kernel


CH = 512
MAIN = (VOCAB // CH) * CH
NCH = MAIN // CH
TAIL0 = VOCAB - WDIM


def _widen_kernel(num_workers):
    chunks_per_w = (NCH + num_workers - 1) // num_workers

    mesh = plsc.VectorSubcoreMesh(core_axis_name="c", subcore_axis_name="s")

    @functools.partial(
        pl.kernel,
        mesh=mesh,
        out_type=jax.ShapeDtypeStruct((VOCAB, WDIM), jnp.float32),
        scratch_types=[
            pltpu.VMEM((DIM, CH), jnp.float32),
            pltpu.VMEM((CH, WDIM), jnp.float32),
            pltpu.VMEM((DIM, WDIM), jnp.float32),
        ],
        compiler_params=pltpu.CompilerParams(needs_layout_passes=False),
    )
    def widen(tabt_hbm, tail_hbm, wide_hbm, vin, vout, vtail):
        num_cores = lax.axis_size("c")
        wid = lax.axis_index("s") * num_cores + lax.axis_index("c")

        rows_q = [
            jax.lax.iota(jnp.int32, L) + L * q for q in range(DIM // L)
        ]

        def transpose_rows(src, n, r0, carry):
            def row(i, c):
                for u in range(4):
                    r = 4 * i + u
                    col = jnp.full((L,), r0 + r, jnp.int32)
                    for q in range(DIM // L):
                        vout[r, pl.ds(q * L, L)] = plsc.load_gather(
                            src, [rows_q[q], col])
                return c
            return lax.fori_loop(0, n // 4, row, carry)

        def body(k, carry):
            chunk = lax.min(wid * chunks_per_w + k, NCH - 1)
            c0 = chunk * CH
            pltpu.sync_copy(tabt_hbm.at[:, pl.ds(c0, CH)], vin)
            carry = transpose_rows(vin, CH, 0, carry)
            pltpu.sync_copy(vout, wide_hbm.at[pl.ds(c0, CH)])
            return carry

        lax.fori_loop(0, chunks_per_w, body, 0)

        @pl.when(wid == 0)
        def _():
            pltpu.sync_copy(tail_hbm, vtail)
            transpose_rows(vtail, WDIM, 0, 0)
            pltpu.sync_copy(vout.at[pl.ds(0, WDIM)],
                            wide_hbm.at[pl.ds(TAIL0, WDIM)])

    return widen


def kernel(indices, table):
    info = plsc.get_sparse_core_info()
    num_workers = info.num_cores * info.num_subcores
    idx_flat = indices.reshape(B)
    tabt = table.T
    wide = _widen_kernel(num_workers)(tabt, tabt[:, TAIL0:])
    out_g = _gather_kernel(num_workers)(idx_flat, wide)
    return out_g.reshape(BATCH, HIST, DIM)


# speedup vs baseline: 3.0320x; 3.0320x over previous
import functools

import jax
import jax.numpy as jnp
from jax import lax
from jax.experimental import pallas as pl
from jax.experimental.pallas import tpu as pltpu
from jax.experimental.pallas import tpu_sc as plsc

VOCAB = 1000000
DIM = 64
WDIM = 128
BATCH = 4096
HIST = 200

B = BATCH * HIST
CHUNK = 128
SUB = 1
GROUP = CHUNK * SUB
L = 16


def _gather_kernel(num_workers):
    b_per_w = B // num_workers
    groups = b_per_w // GROUP
    pairs = groups // 2

    mesh = plsc.VectorSubcoreMesh(core_axis_name="c", subcore_axis_name="s")

    @functools.partial(
        pl.kernel,
        mesh=mesh,
        out_type=jax.ShapeDtypeStruct((B, DIM), jnp.float32),
        scratch_types=[
            pltpu.VMEM((b_per_w,), jnp.int32),
            pltpu.VMEM((GROUP, WDIM), jnp.float32),
            pltpu.VMEM((GROUP, WDIM), jnp.float32),
            pltpu.VMEM((GROUP, DIM), jnp.float32),
            pltpu.VMEM((GROUP, DIM), jnp.float32),
            pltpu.SemaphoreType.DMA,
            pltpu.SemaphoreType.DMA,
            pltpu.SemaphoreType.DMA,
            pltpu.SemaphoreType.DMA,
        ],
    )
    def gather_kernel(idx_hbm, wide_hbm, out_hbm, idx_v, rows0, rows1,
                      nar0, nar1, sem_g0, sem_g1, sem_o0, sem_o1):
        num_cores = lax.axis_size("c")
        wid = lax.axis_index("s") * num_cores + lax.axis_index("c")
        row_base = wid * b_per_w

        pltpu.sync_copy(idx_hbm.at[pl.ds(row_base, b_per_w)], idx_v)

        def fire_gather(g, rows_v, sem):
            for j in range(SUB):
                pltpu.async_copy(
                    wide_hbm.at[idx_v.at[pl.ds(g * GROUP + j * CHUNK, CHUNK)]],
                    rows_v.at[pl.ds(j * CHUNK, CHUNK)],
                    sem,
                )

        def wait_gather(rows_v, sem):
            pltpu.make_async_copy(wide_hbm.at[pl.ds(0, GROUP)], rows_v, sem).wait()

        def wait_out(nar_v, sem):
            pltpu.make_async_copy(
                out_hbm.at[pl.ds(0, GROUP)], nar_v, sem).wait()

        def compact(rows_v, nar_v):
            def body(i, carry):
                for u in range(16):
                    r = 16 * i + u
                    for q in range(DIM // L):
                        nar_v[r, pl.ds(q * L, L)] = rows_v[r, pl.ds(q * L, L)]
                return carry
            lax.fori_loop(0, GROUP // 16, body, 0)

        def fire_out(g, nar_v, sem):
            pltpu.async_copy(
                nar_v, out_hbm.at[pl.ds(row_base + g * GROUP, GROUP)], sem
            )

        fire_gather(0, rows0, sem_g0)

        def body(k, carry):
            g0 = 2 * k

            fire_gather(g0 + 1, rows1, sem_g1)

            wait_gather(rows0, sem_g0)

            @pl.when(k > 0)
            def _():
                wait_out(nar0, sem_o0)

            compact(rows0, nar0)
            fire_out(g0, nar0, sem_o0)

            g_next = lax.min(g0 + 2, groups - 1)
            fire_gather(g_next, rows0, sem_g0)

            wait_gather(rows1, sem_g1)

            @pl.when(k > 0)
            def _():
                wait_out(nar1, sem_o1)

            compact(rows1, nar1)
            fire_out(g0 + 1, nar1, sem_o1)
            return carry

        lax.fori_loop(0, pairs, body, 0)

        wait_gather(rows0, sem_g0)
        wait_out(nar0, sem_o0)
        wait_out(nar1, sem_o1)

    return gather_kernel


TBLK = 8192


def _widen_kernel():
    grid = (VOCAB + TBLK - 1) // TBLK

    def body(in_ref, out_ref):
        x = in_ref[...]
        out_ref[:, :DIM] = x.T

    return pl.pallas_call(
        body,
        grid=(grid,),
        in_specs=[pl.BlockSpec((DIM, TBLK), lambda i: (0, i))],
        out_specs=pl.BlockSpec((TBLK, WDIM), lambda i: (i, 0)),
        out_shape=jax.ShapeDtypeStruct((grid * TBLK, WDIM), jnp.float32),
    )


def kernel(indices, table):
    info = plsc.get_sparse_core_info()
    num_workers = info.num_cores * info.num_subcores
    idx_flat = indices.reshape(B)
    wide = _widen_kernel()(table.T)
    out_g = _gather_kernel(num_workers)(idx_flat, wide)
    return out_g.reshape(BATCH, HIST, DIM)


# speedup vs baseline: 3.1000x; 1.0224x over previous
import functools

import jax
import jax.numpy as jnp
from jax import lax
from jax.experimental import pallas as pl
from jax.experimental.pallas import tpu as pltpu
from jax.experimental.pallas import tpu_sc as plsc

VOCAB = 1000000
DIM = 64
WDIM = 128
BATCH = 4096
HIST = 200

B = BATCH * HIST
CHUNK = 128
SUB = 1
GROUP = CHUNK * SUB
L = 16


def _gather_kernel(num_workers):
    b_per_w = B // num_workers
    groups = b_per_w // GROUP
    pairs = groups // 2

    mesh = plsc.VectorSubcoreMesh(core_axis_name="c", subcore_axis_name="s")

    @functools.partial(
        pl.kernel,
        mesh=mesh,
        out_type=jax.ShapeDtypeStruct((B, DIM), jnp.float32),
        scratch_types=[
            pltpu.VMEM((b_per_w,), jnp.int32),
            pltpu.VMEM((GROUP, WDIM), jnp.float32),
            pltpu.VMEM((GROUP, WDIM), jnp.float32),
            pltpu.VMEM((GROUP, DIM), jnp.float32),
            pltpu.VMEM((GROUP, DIM), jnp.float32),
            pltpu.SemaphoreType.DMA,
            pltpu.SemaphoreType.DMA,
            pltpu.SemaphoreType.DMA,
            pltpu.SemaphoreType.DMA,
        ],
    )
    def gather_kernel(idx_hbm, wide_hbm, out_hbm, idx_v, rows0, rows1,
                      nar0, nar1, sem_g0, sem_g1, sem_o0, sem_o1):
        num_cores = lax.axis_size("c")
        wid = lax.axis_index("s") * num_cores + lax.axis_index("c")
        row_base = wid * b_per_w

        pltpu.sync_copy(idx_hbm.at[pl.ds(row_base, b_per_w)], idx_v)

        def fire_gather(g, rows_v, sem):
            for j in range(SUB):
                pltpu.async_copy(
                    wide_hbm.at[idx_v.at[pl.ds(g * GROUP + j * CHUNK, CHUNK)]],
                    rows_v.at[pl.ds(j * CHUNK, CHUNK)],
                    sem,
                )

        def wait_gather(rows_v, sem):
            pltpu.make_async_copy(wide_hbm.at[pl.ds(0, GROUP)], rows_v, sem).wait()

        def wait_out(nar_v, sem):
            pltpu.make_async_copy(
                out_hbm.at[pl.ds(0, GROUP)], nar_v, sem).wait()

        def compact(rows_v, nar_v):
            def body(i, carry):
                for u in range(16):
                    r = 16 * i + u
                    for q in range(DIM // L):
                        nar_v[r, pl.ds(q * L, L)] = rows_v[r, pl.ds(q * L, L)]
                return carry
            lax.fori_loop(0, GROUP // 16, body, 0)

        def fire_out(g, nar_v, sem):
            pltpu.async_copy(
                nar_v, out_hbm.at[pl.ds(row_base + g * GROUP, GROUP)], sem
            )

        fire_gather(0, rows0, sem_g0)

        def body(k, carry):
            g0 = 2 * k

            fire_gather(g0 + 1, rows1, sem_g1)

            wait_gather(rows0, sem_g0)

            @pl.when(k > 0)
            def _():
                wait_out(nar0, sem_o0)

            compact(rows0, nar0)
            fire_out(g0, nar0, sem_o0)

            g_next = lax.min(g0 + 2, groups - 1)
            fire_gather(g_next, rows0, sem_g0)

            wait_gather(rows1, sem_g1)

            @pl.when(k > 0)
            def _():
                wait_out(nar1, sem_o1)

            compact(rows1, nar1)
            fire_out(g0 + 1, nar1, sem_o1)
            return carry

        lax.fori_loop(0, pairs, body, 0)

        wait_gather(rows0, sem_g0)
        wait_out(nar0, sem_o0)
        wait_out(nar1, sem_o1)

    return gather_kernel


TBLK = 16384


def _widen_kernel():
    grid = (VOCAB + TBLK - 1) // TBLK

    def body(in_ref, out_ref):
        x = in_ref[...]
        out_ref[:, :DIM] = x.T

    return pl.pallas_call(
        body,
        grid=(grid,),
        in_specs=[pl.BlockSpec((DIM, TBLK), lambda i: (0, i))],
        out_specs=pl.BlockSpec((TBLK, WDIM), lambda i: (i, 0)),
        out_shape=jax.ShapeDtypeStruct((grid * TBLK, WDIM), jnp.float32),
    )


def kernel(indices, table):
    info = plsc.get_sparse_core_info()
    num_workers = info.num_cores * info.num_subcores
    idx_flat = indices.reshape(B)
    wide = _widen_kernel()(table.T)
    out_g = _gather_kernel(num_workers)(idx_flat, wide)
    return out_g.reshape(BATCH, HIST, DIM)
